# Initial kernel scaffold; baseline (speedup 1.0000x reference)
import functools

import jax
import jax.numpy as jnp
from jax.experimental import pallas as pl
from jax.experimental.pallas import tpu as pltpu


def _tok_kernel(tok_ref, cm_ref, vecs32_ref, vecs128_ref, pos_ref, woT_ref,
                out_ref, tab_ref, *, max_cols, hid, q):
    @pl.when(pl.program_id(0) == 0)
    def _fold():
        sm = vecs32_ref[:]
        v128 = vecs128_ref[:]
        woT = woT_ref[:]
        f32 = jnp.float32
        wv_row = jnp.dot(sm[0:1, :], woT[0:q, :], preferred_element_type=f32)
        bv_row = jnp.dot(sm[1:2, :], woT[0:q, :], preferred_element_type=f32)
        obs_proj = jnp.dot(sm[2:4, :], woT[q:2 * q, :], preferred_element_type=f32)
        mask_proj = jnp.dot(sm[4:6, :], woT[2 * q:3 * q, :], preferred_element_type=f32)
        pos_proj = jnp.dot(pos_ref[:], woT[3 * q:4 * q, :], preferred_element_type=f32)
        const_row = v128[0:1, :] + bv_row + obs_proj[0:1, :] + mask_proj[0:1, :]
        tab_ref[0:1, :] = wv_row
        tab_ref[1:2, :] = const_row
        tab_ref[2:3, :] = obs_proj[1:2, :] - obs_proj[0:1, :]
        tab_ref[3:4, :] = mask_proj[1:2, :] - mask_proj[0:1, :]
        tab_ref[4:5, :] = v128[1:2, :]
        tab_ref[5:6, :] = v128[2:3, :]
        tab_ref[8:8 + max_cols, :] = pos_proj

    tok = tok_ref[:]
    tab = tab_ref[:]
    v = tok[:, 0:1]
    o = tok[:, 1:2]
    m = tok[:, 2:3]
    f = tok[:, 3:4]
    s = v * o
    oi = jnp.clip(jnp.floor(o), 0.0, 1.0)
    mi = jnp.clip(jnp.floor(m), 0.0, 1.0)
    fi = jnp.clip(jnp.floor(f * (max_cols - 1)), 0.0, max_cols - 1)
    t = tok.shape[0]
    lane = jax.lax.broadcasted_iota(jnp.float32, (t, max_cols), 1)
    onehot = (fi == lane).astype(jnp.float32)
    pos_sel = jnp.dot(onehot, tab[8:8 + max_cols, :],
                      preferred_element_type=jnp.float32)
    acc = (pos_sel + s * tab[0:1, :] + tab[1:2, :]
           + oi * tab[2:3, :] + mi * tab[3:4, :])
    mu = jnp.mean(acc, axis=1, keepdims=True)
    d = acc - mu
    var = jnp.mean(d * d, axis=1, keepdims=True)
    y = d * jax.lax.rsqrt(var + 1e-5) * tab[4:5, :] + tab[5:6, :]
    out_ref[:] = y * cm_ref[:]


def kernel(tokens, Wv, bv, obs_table, mask_table, pos_table, Wo, bo, gamma,
           beta, col_mask):
    B, R, C, _ = tokens.shape
    HID = Wo.shape[0]
    Q = Wv.shape[0]
    MAX_COLS = pos_table.shape[0]
    N = B * R * C
    T = 8192
    grid = (N // T,)

    tok2 = tokens.reshape(N, 4)
    cm_tok = jnp.broadcast_to(col_mask[:, None, :], (B, R, C)) \
        .astype(jnp.float32).reshape(N, 1)
    vecs32 = jnp.concatenate([
        Wv.reshape(1, Q), bv.reshape(1, Q), obs_table, mask_table,
        jnp.zeros((2, Q), jnp.float32)], axis=0)
    vecs128 = jnp.concatenate([
        bo.reshape(1, HID), gamma.reshape(1, HID), beta.reshape(1, HID),
        jnp.zeros((5, HID), jnp.float32)], axis=0)
    woT = Wo.T

    out = pl.pallas_call(
        functools.partial(_tok_kernel, max_cols=MAX_COLS, hid=HID, q=Q),
        grid=grid,
        in_specs=[
            pl.BlockSpec((T, 4), lambda i: (i, 0)),
            pl.BlockSpec((T, 1), lambda i: (i, 0)),
            pl.BlockSpec((8, Q), lambda i: (0, 0)),
            pl.BlockSpec((8, HID), lambda i: (0, 0)),
            pl.BlockSpec((MAX_COLS, Q), lambda i: (0, 0)),
            pl.BlockSpec((4 * Q, HID), lambda i: (0, 0)),
        ],
        out_specs=pl.BlockSpec((T, HID), lambda i: (i, 0)),
        out_shape=jax.ShapeDtypeStruct((N, HID), jnp.float32),
        scratch_shapes=[pltpu.VMEM((8 + MAX_COLS, HID), jnp.float32)],
        compiler_params=pltpu.CompilerParams(
            dimension_semantics=("arbitrary",)),
    )(tok2, cm_tok, vecs32, vecs128, pos_table, woT)
    return out.reshape(B, R, C, HID)


# speedup vs baseline: 5.5160x; 5.5160x over previous
import functools

import jax
import jax.numpy as jnp
from jax.experimental import pallas as pl
from jax.experimental.pallas import tpu as pltpu


def _tok_kernel(tok_ref, cm_ref, vecs32_ref, vecs128_ref, pos_ref, woT_ref,
                out_ref, tab_ref, *, max_cols, hid, q):
    @pl.when(pl.program_id(0) == 0)
    def _fold():
        sm = vecs32_ref[:]
        v128 = vecs128_ref[:]
        woT = woT_ref[:]
        f32 = jnp.float32
        wv_row = jnp.dot(sm[0:1, :], woT[0:q, :], preferred_element_type=f32)
        bv_row = jnp.dot(sm[1:2, :], woT[0:q, :], preferred_element_type=f32)
        obs_proj = jnp.dot(sm[2:4, :], woT[q:2 * q, :], preferred_element_type=f32)
        mask_proj = jnp.dot(sm[4:6, :], woT[2 * q:3 * q, :], preferred_element_type=f32)
        pos_proj = jnp.dot(pos_ref[:], woT[3 * q:4 * q, :], preferred_element_type=f32)
        const_row = v128[0:1, :] + bv_row + obs_proj[0:1, :] + mask_proj[0:1, :]
        tab_ref[0:1, :] = wv_row
        tab_ref[1:2, :] = const_row
        tab_ref[2:3, :] = obs_proj[1:2, :] - obs_proj[0:1, :]
        tab_ref[3:4, :] = mask_proj[1:2, :] - mask_proj[0:1, :]
        tab_ref[4:5, :] = v128[1:2, :]
        tab_ref[5:6, :] = v128[2:3, :]
        tab_ref[8:8 + max_cols, :] = pos_proj

    tok = tok_ref[:]
    tab = tab_ref[:]
    v = tok[:, 0:1]
    o = tok[:, 1:2]
    m = tok[:, 2:3]
    f = tok[:, 3:4]
    s = v * o
    oi = jnp.clip(jnp.floor(o), 0.0, 1.0)
    mi = jnp.clip(jnp.floor(m), 0.0, 1.0)
    fi = jnp.clip(jnp.floor(f * (max_cols - 1)), 0.0, max_cols - 1) \
        .astype(jnp.int32)
    t = tok.shape[0]
    lane = jax.lax.broadcasted_iota(jnp.int32, (t, max_cols), 1)
    onehot = (fi == lane).astype(jnp.float32)
    pos_sel = jnp.dot(onehot, tab[8:8 + max_cols, :],
                      preferred_element_type=jnp.float32)
    acc = (pos_sel + s * tab[0:1, :] + tab[1:2, :]
           + oi * tab[2:3, :] + mi * tab[3:4, :])
    mu = jnp.mean(acc, axis=1, keepdims=True)
    d = acc - mu
    var = jnp.mean(d * d, axis=1, keepdims=True)
    y = d * jax.lax.rsqrt(var + 1e-5) * tab[4:5, :] + tab[5:6, :]
    out_ref[:] = y * cm_ref[:]


def kernel(tokens, Wv, bv, obs_table, mask_table, pos_table, Wo, bo, gamma,
           beta, col_mask):
    B, R, C, _ = tokens.shape
    HID = Wo.shape[0]
    Q = Wv.shape[0]
    MAX_COLS = pos_table.shape[0]
    N = B * R * C
    T = 8192
    grid = (N // T,)

    tok2 = tokens.reshape(N, 4)
    cm_tok = jnp.broadcast_to(col_mask[:, None, :], (B, R, C)) \
        .astype(jnp.float32).reshape(N, 1)
    vecs32 = jnp.concatenate([
        Wv.reshape(1, Q), bv.reshape(1, Q), obs_table, mask_table,
        jnp.zeros((2, Q), jnp.float32)], axis=0)
    vecs128 = jnp.concatenate([
        bo.reshape(1, HID), gamma.reshape(1, HID), beta.reshape(1, HID),
        jnp.zeros((5, HID), jnp.float32)], axis=0)
    woT = Wo.T

    out = pl.pallas_call(
        functools.partial(_tok_kernel, max_cols=MAX_COLS, hid=HID, q=Q),
        grid=grid,
        in_specs=[
            pl.BlockSpec((T, 4), lambda i: (i, 0)),
            pl.BlockSpec((T, 1), lambda i: (i, 0)),
            pl.BlockSpec((8, Q), lambda i: (0, 0)),
            pl.BlockSpec((8, HID), lambda i: (0, 0)),
            pl.BlockSpec((MAX_COLS, Q), lambda i: (0, 0)),
            pl.BlockSpec((4 * Q, HID), lambda i: (0, 0)),
        ],
        out_specs=pl.BlockSpec((T, HID), lambda i: (i, 0)),
        out_shape=jax.ShapeDtypeStruct((N, HID), jnp.float32),
        scratch_shapes=[pltpu.VMEM((8 + MAX_COLS, HID), jnp.float32)],
        compiler_params=pltpu.CompilerParams(
            dimension_semantics=("arbitrary",)),
    )(tok2, cm_tok, vecs32, vecs128, pos_table, woT)
    return out.reshape(B, R, C, HID)


# speedup vs baseline: 6.4630x; 1.1717x over previous
import functools

import jax
import jax.numpy as jnp
from jax.experimental import pallas as pl
from jax.experimental.pallas import tpu as pltpu


def _tok_kernel(tok_ref, cm_ref, vecs32_ref, vecs128_ref, pos_ref, woT_ref,
                out_ref, tab_ref, *, max_cols, hid, q, t):
    f32 = jnp.float32

    @pl.when(pl.program_id(0) == 0)
    def _fold():
        sm = vecs32_ref[:]
        v128 = vecs128_ref[:]
        woT = woT_ref[:]

        def center(r):
            return r - jnp.mean(r, axis=1, keepdims=True)

        wv_row = jnp.dot(sm[0:1, :], woT[0:q, :], preferred_element_type=f32)
        bv_row = jnp.dot(sm[1:2, :], woT[0:q, :], preferred_element_type=f32)
        obs_proj = jnp.dot(sm[2:4, :], woT[q:2 * q, :],
                           preferred_element_type=f32)
        mask_proj = jnp.dot(sm[4:6, :], woT[2 * q:3 * q, :],
                            preferred_element_type=f32)
        pos_proj = jnp.dot(pos_ref[:], woT[3 * q:4 * q, :],
                           preferred_element_type=f32)
        const_row = v128[0:1, :] + bv_row + obs_proj[0:1, :] + mask_proj[0:1, :]
        tab_ref[0:max_cols, :] = center(pos_proj)
        tab_ref[max_cols:max_cols + 1, :] = center(wv_row)
        tab_ref[max_cols + 1:max_cols + 2, :] = center(const_row)
        tab_ref[max_cols + 2:max_cols + 3, :] = \
            center(obs_proj[1:2, :] - obs_proj[0:1, :])
        tab_ref[max_cols + 3:max_cols + 4, :] = \
            center(mask_proj[1:2, :] - mask_proj[0:1, :])
        tab_ref[max_cols + 4:max_cols + 8, :] = jnp.zeros((4, hid), f32)

    tok = tok_ref[:]
    kw = max_cols + 8

    l4 = jax.lax.broadcasted_iota(jnp.int32, (1, 4), 1)
    mult4 = jnp.where(l4 == 3, float(max_cols - 1), 1.0)
    cap4 = jnp.where(l4 == 3, float(max_cols - 1), 1.0)
    w = jnp.clip(jnp.floor(tok * mult4), 0.0, cap4)
    s = tok[:, 0:1] * tok[:, 1:2]
    oi = w[:, 1:2]
    mi = w[:, 2:3]
    fi = w[:, 3:4].astype(jnp.int32)

    li = jax.lax.broadcasted_iota(jnp.int32, (t, kw), 1)
    coef = (li == fi).astype(f32)
    coef = jnp.where(li == max_cols, s, coef)
    coef = jnp.where(li == max_cols + 1, 1.0, coef)
    coef = jnp.where(li == max_cols + 2, oi, coef)
    coef = jnp.where(li == max_cols + 3, mi, coef)

    d = jnp.dot(coef, tab_ref[:], preferred_element_type=f32)
    ssq = jnp.dot(d * d, jnp.full((hid, hid), 1.0 / hid, f32),
                  preferred_element_type=f32)
    scale = jax.lax.rsqrt(ssq + 1e-5)
    v128 = vecs128_ref[:]
    y = d * scale * v128[1:2, :] + v128[2:3, :]
    y3 = y.reshape(t // max_cols, max_cols, hid)
    out_ref[:] = (y3 * cm_ref[:]).reshape(t, hid)


def kernel(tokens, Wv, bv, obs_table, mask_table, pos_table, Wo, bo, gamma,
           beta, col_mask):
    B, R, C, _ = tokens.shape
    HID = Wo.shape[0]
    Q = Wv.shape[0]
    MAX_COLS = pos_table.shape[0]
    N = B * R * C
    T = R * C
    grid = (B,)

    tok2 = tokens.reshape(N, 4)
    cmf = col_mask.astype(jnp.float32).reshape(B, C, 1)
    vecs32 = jnp.concatenate([
        Wv.reshape(1, Q), bv.reshape(1, Q), obs_table, mask_table,
        jnp.zeros((2, Q), jnp.float32)], axis=0)
    vecs128 = jnp.concatenate([
        bo.reshape(1, HID), gamma.reshape(1, HID), beta.reshape(1, HID),
        jnp.zeros((5, HID), jnp.float32)], axis=0)
    woT = Wo.T

    out = pl.pallas_call(
        functools.partial(_tok_kernel, max_cols=MAX_COLS, hid=HID, q=Q, t=T),
        grid=grid,
        in_specs=[
            pl.BlockSpec((T, 4), lambda i: (i, 0)),
            pl.BlockSpec((1, C, 1), lambda i: (i, 0, 0)),
            pl.BlockSpec((8, Q), lambda i: (0, 0)),
            pl.BlockSpec((8, HID), lambda i: (0, 0)),
            pl.BlockSpec((MAX_COLS, Q), lambda i: (0, 0)),
            pl.BlockSpec((4 * Q, HID), lambda i: (0, 0)),
        ],
        out_specs=pl.BlockSpec((T, HID), lambda i: (i, 0)),
        out_shape=jax.ShapeDtypeStruct((N, HID), jnp.float32),
        scratch_shapes=[pltpu.VMEM((MAX_COLS + 8, HID), jnp.float32)],
        compiler_params=pltpu.CompilerParams(
            dimension_semantics=("arbitrary",)),
    )(tok2, cmf, vecs32, vecs128, pos_table, woT)
    return out.reshape(B, R, C, HID)


# speedup vs baseline: 32.7735x; 5.0709x over previous
import functools

import jax
import jax.numpy as jnp
from jax.experimental import pallas as pl
from jax.experimental.pallas import tpu as pltpu


def _tok_kernel(tok_ref, cm_ref, vecs32_ref, vecs128_ref, pos_ref, woT_ref,
                out_ref, tab_ref, *, max_cols, hid, q, t):
    f32 = jnp.float32

    @pl.when(pl.program_id(0) == 0)
    def _fold():
        sm = vecs32_ref[:]
        v128 = vecs128_ref[:]
        woT = woT_ref[:]

        def center(r):
            return r - jnp.mean(r, axis=1, keepdims=True)

        wv_row = jnp.dot(sm[0:1, :], woT[0:q, :], preferred_element_type=f32)
        bv_row = jnp.dot(sm[1:2, :], woT[0:q, :], preferred_element_type=f32)
        obs_proj = jnp.dot(sm[2:4, :], woT[q:2 * q, :],
                           preferred_element_type=f32)
        mask_proj = jnp.dot(sm[4:6, :], woT[2 * q:3 * q, :],
                            preferred_element_type=f32)
        pos_proj = jnp.dot(pos_ref[:], woT[3 * q:4 * q, :],
                           preferred_element_type=f32)
        const_row = v128[0:1, :] + bv_row + obs_proj[0:1, :] + mask_proj[0:1, :]
        tab_ref[0:max_cols, :] = center(pos_proj)
        tab_ref[max_cols:max_cols + 1, :] = center(wv_row)
        tab_ref[max_cols + 1:max_cols + 2, :] = center(const_row)
        tab_ref[max_cols + 2:max_cols + 3, :] = \
            center(obs_proj[1:2, :] - obs_proj[0:1, :])
        tab_ref[max_cols + 3:max_cols + 4, :] = \
            center(mask_proj[1:2, :] - mask_proj[0:1, :])
        tab_ref[max_cols + 4:max_cols + 8, :] = jnp.zeros((4, hid), f32)

    tt = tok_ref[:]

    r4 = jax.lax.broadcasted_iota(jnp.int32, (4, 1), 0)
    mult4 = jnp.where(r4 == 3, float(max_cols - 1), 1.0)
    cap4 = jnp.where(r4 == 3, float(max_cols - 1), 1.0)
    w = jnp.clip(jnp.floor(tt * mult4), 0.0, cap4)
    s_row = tt[0:1, :] * tt[1:2, :]
    fi_row = w[3:4, :].astype(jnp.int32)

    ri = jax.lax.broadcasted_iota(jnp.int32, (max_cols, t), 0)
    onehotT = (ri == fi_row).astype(f32)
    coefT = jnp.concatenate([
        onehotT,
        s_row,
        jnp.ones((1, t), f32),
        w[1:2, :],
        w[2:3, :],
        jnp.zeros((4, t), f32),
    ], axis=0)

    d = jax.lax.dot_general(
        coefT, tab_ref[:],
        dimension_numbers=(((0,), (0,)), ((), ())),
        preferred_element_type=f32)
    ssq = jnp.dot(d * d, jnp.full((hid, hid), 1.0 / hid, f32),
                  preferred_element_type=f32)
    scale = jax.lax.rsqrt(ssq + 1e-5)
    v128 = vecs128_ref[:]
    y = d * scale * v128[1:2, :] + v128[2:3, :]
    y3 = y.reshape(t // max_cols, max_cols, hid)
    out_ref[:] = (y3 * cm_ref[:]).reshape(t, hid)


def kernel(tokens, Wv, bv, obs_table, mask_table, pos_table, Wo, bo, gamma,
           beta, col_mask):
    B, R, C, _ = tokens.shape
    HID = Wo.shape[0]
    Q = Wv.shape[0]
    MAX_COLS = pos_table.shape[0]
    N = B * R * C
    T = R * C
    grid = (B,)

    tok_t = tokens.reshape(N, 4).T
    cmf = col_mask.astype(jnp.float32).reshape(B, C, 1)
    vecs32 = jnp.concatenate([
        Wv.reshape(1, Q), bv.reshape(1, Q), obs_table, mask_table,
        jnp.zeros((2, Q), jnp.float32)], axis=0)
    vecs128 = jnp.concatenate([
        bo.reshape(1, HID), gamma.reshape(1, HID), beta.reshape(1, HID),
        jnp.zeros((5, HID), jnp.float32)], axis=0)
    woT = Wo.T

    out = pl.pallas_call(
        functools.partial(_tok_kernel, max_cols=MAX_COLS, hid=HID, q=Q, t=T),
        grid=grid,
        in_specs=[
            pl.BlockSpec((4, T), lambda i: (0, i)),
            pl.BlockSpec((1, C, 1), lambda i: (i, 0, 0)),
            pl.BlockSpec((8, Q), lambda i: (0, 0)),
            pl.BlockSpec((8, HID), lambda i: (0, 0)),
            pl.BlockSpec((MAX_COLS, Q), lambda i: (0, 0)),
            pl.BlockSpec((4 * Q, HID), lambda i: (0, 0)),
        ],
        out_specs=pl.BlockSpec((T, HID), lambda i: (i, 0)),
        out_shape=jax.ShapeDtypeStruct((N, HID), jnp.float32),
        scratch_shapes=[pltpu.VMEM((MAX_COLS + 8, HID), jnp.float32)],
        compiler_params=pltpu.CompilerParams(
            dimension_semantics=("arbitrary",)),
    )(tok_t, cmf, vecs32, vecs128, pos_table, woT)
    return out.reshape(B, R, C, HID)
